# baseline (speedup 1.0000x reference)
import functools

import jax
import jax.numpy as jnp
from jax import lax
from jax.experimental import pallas as pl
from jax.experimental.pallas import tpu as pltpu
from jax.experimental.pallas import tpu_sc as plsc

NC = 2
NS = 16
NW = NC * NS
NB = 5
LANE = 128
SUB = 8


@functools.cache
def _build(batch_sz: int, hist: int, vocab: int, dim: int):
    assert batch_sz % (NW * LANE) == 0 and batch_sz // NW == LANE
    assert dim % SUB == 0
    bw = batch_sz // NW
    dt_n = dim // SUB
    assert hist % NB == 0 and hist >= 2 * NB
    n_grp = hist // NB

    def body(table_hbm, batch_hbm, out_hbm, idx_v, idx_t, sbuf, ssem, *rest):
        bufs = rest[:NB]
        tbufs = rest[NB:2 * NB]
        gsems = rest[2 * NB:3 * NB]
        osems = rest[3 * NB:]
        K = NB - 1
        wid = lax.axis_index("s") * NC + lax.axis_index("c")

        pltpu.sync_copy(batch_hbm.at[pl.ds(wid * bw, bw)], idx_v)

        @pl.loop(0, hist)
        def _(h):
            cols = jnp.full((16,), 0, jnp.int32) + h
            for k in range(bw // 16):
                rows = lax.iota(jnp.int32, 16) + k * 16
                v = plsc.load_gather(idx_v, [rows, cols])
                idx_t[h, pl.ds(k * 16, 16)] = v

        def gather(h, slot):
            pltpu.async_copy(
                table_hbm.at[idx_t.at[h]], bufs[slot], gsems[slot])

        for j in range(K):
            gather(j, j)

        @pl.loop(0, n_grp)
        def _(g):
            for b in range(NB):
                j = g * NB + b
                pltpu.make_async_copy(
                    table_hbm.at[idx_t.at[b]], bufs[b], gsems[b]).wait()

                @pl.when(j + K < hist)
                def _():
                    gather(j + K, (b + K) % NB)

                @pl.when(g > 0)
                def _():
                    for dt in range(dt_n):
                        pltpu.make_async_copy(
                            tbufs[b].at[dt], out_hbm.at[0, dt, wid],
                            osems[b]).wait()

                @plsc.parallel_loop(0, bw, step=4, unroll=4)
                def _(ri):
                    for rr in range(4):
                        r = ri + rr
                        for k in range(dim // 16):
                            sbuf[r, pl.ds(k * 16, 16)] = (
                                bufs[b][r, pl.ds(k * 16, 16)])

                @plsc.parallel_loop(0, dt_n, unroll=2)
                def _(dt):
                    for ds in range(SUB):
                        cols = jnp.full((16,), 0, jnp.int32) + (dt * SUB + ds)
                        for k in range(bw // 16):
                            rows = lax.iota(jnp.int32, 16) + k * 16
                            v = plsc.load_gather(sbuf, [rows, cols])
                            tbufs[b][dt, ds, pl.ds(k * 16, 16)] = v

                for dt in range(dt_n):
                    pltpu.async_copy(
                        tbufs[b].at[dt], out_hbm.at[j, dt, wid], osems[b])

        for b in range(NB):
            for dt in range(dt_n):
                pltpu.make_async_copy(
                    tbufs[b].at[dt], out_hbm.at[0, dt, wid], osems[b]).wait()

    return pl.kernel(
        body,
        out_type=jax.ShapeDtypeStruct(
            (hist, dt_n, NW, SUB, LANE), jnp.float32),
        mesh=plsc.VectorSubcoreMesh(core_axis_name="c", subcore_axis_name="s"),
        scratch_types=[
            pltpu.VMEM((bw, hist), jnp.int32),
            pltpu.VMEM((hist, bw), jnp.int32),
            pltpu.VMEM((bw, dim + 1), jnp.float32),
            pltpu.SemaphoreType.DMA,
            *[pltpu.VMEM((bw, dim), jnp.float32) for _ in range(NB)],
            *[pltpu.VMEM((dt_n, SUB, bw), jnp.float32) for _ in range(NB)],
            *[pltpu.SemaphoreType.DMA for _ in range(2 * NB)],
        ],
        compiler_params=pltpu.CompilerParams(
            use_tc_tiling_on_sc=False, needs_layout_passes=False),
    )


def kernel(batch, weight):
    batch_sz, hist = batch.shape
    vocab, dim = weight.shape
    t5 = _build(batch_sz, hist, vocab, dim)(weight, batch)
    return (t5.transpose(2, 4, 0, 1, 3)
            .reshape(batch_sz, hist, dim))


# speedup vs baseline: 1.0795x; 1.0795x over previous
import functools

import jax
import jax.numpy as jnp
from jax import lax
from jax.experimental import pallas as pl
from jax.experimental.pallas import tpu as pltpu
from jax.experimental.pallas import tpu_sc as plsc

NC = 2
NS = 16
NW = NC * NS
NB = 5
LANE = 128
SUB = 8


@functools.cache
def _build(batch_sz: int, hist: int, vocab: int, dim: int):
    assert batch_sz % (NW * LANE) == 0 and batch_sz // NW == LANE
    assert dim % SUB == 0
    bw = batch_sz // NW
    dt_n = dim // SUB
    assert hist % NB == 0 and hist >= 2 * NB
    n_grp = hist // NB

    def body(table_hbm, batch_hbm, out_hbm, idx_v, idx_t, sbuf, ssem, *rest):
        bufs = rest[:NB]
        tbufs = rest[NB:2 * NB]
        gsems = rest[2 * NB:3 * NB]
        osems = rest[3 * NB:]
        K = NB - 1
        wid = lax.axis_index("s") * NC + lax.axis_index("c")

        pltpu.sync_copy(batch_hbm.at[pl.ds(wid * bw, bw)], idx_v)

        @pl.loop(0, hist)
        def _(h):
            cols = jnp.full((16,), 0, jnp.int32) + h
            for k in range(bw // 16):
                rows = lax.iota(jnp.int32, 16) + k * 16
                v = plsc.load_gather(idx_v, [rows, cols])
                idx_t[h, pl.ds(k * 16, 16)] = v

        def gather(h, slot):
            pltpu.async_copy(
                table_hbm.at[idx_t.at[h]], bufs[slot], gsems[slot])

        for j in range(K):
            gather(j, j)

        @pl.loop(0, n_grp)
        def _(g):
            for b in range(NB):
                j = g * NB + b
                pltpu.make_async_copy(
                    table_hbm.at[idx_t.at[b]], bufs[b], gsems[b]).wait()

                @pl.when(j + K < hist)
                def _():
                    gather(j + K, (b + K) % NB)

                @pl.when(g > 0)
                def _():
                    for dt in range(dt_n):
                        pltpu.make_async_copy(
                            tbufs[b].at[dt], out_hbm.at[0, dt, wid],
                            osems[b]).wait()

                @plsc.parallel_loop(0, bw, step=4)
                def _(ri):
                    for rr in range(4):
                        r = ri + rr
                        for k in range(dim // 16):
                            sbuf[r, pl.ds(k * 16, 16)] = (
                                bufs[b][r, pl.ds(k * 16, 16)])

                @plsc.parallel_loop(0, dt_n)
                def _(dt):
                    for ds in range(SUB):
                        cols = jnp.full((16,), 0, jnp.int32) + (dt * SUB + ds)
                        for k in range(bw // 16):
                            rows = lax.iota(jnp.int32, 16) + k * 16
                            v = plsc.load_gather(sbuf, [rows, cols])
                            tbufs[b][dt, ds, pl.ds(k * 16, 16)] = v

                for dt in range(dt_n):
                    pltpu.async_copy(
                        tbufs[b].at[dt], out_hbm.at[j, dt, wid], osems[b])

        for b in range(NB):
            for dt in range(dt_n):
                pltpu.make_async_copy(
                    tbufs[b].at[dt], out_hbm.at[0, dt, wid], osems[b]).wait()

    return pl.kernel(
        body,
        out_type=jax.ShapeDtypeStruct(
            (hist, dt_n, NW, SUB, LANE), jnp.float32),
        mesh=plsc.VectorSubcoreMesh(core_axis_name="c", subcore_axis_name="s"),
        scratch_types=[
            pltpu.VMEM((bw, hist), jnp.int32),
            pltpu.VMEM((hist, bw), jnp.int32),
            pltpu.VMEM((bw, dim + 1), jnp.float32),
            pltpu.SemaphoreType.DMA,
            *[pltpu.VMEM((bw, dim), jnp.float32) for _ in range(NB)],
            *[pltpu.VMEM((dt_n, SUB, bw), jnp.float32) for _ in range(NB)],
            *[pltpu.SemaphoreType.DMA for _ in range(2 * NB)],
        ],
        compiler_params=pltpu.CompilerParams(
            use_tc_tiling_on_sc=False, needs_layout_passes=False),
    )


def kernel(batch, weight):
    batch_sz, hist = batch.shape
    vocab, dim = weight.shape
    t5 = _build(batch_sz, hist, vocab, dim)(weight, batch)
    return (t5.transpose(2, 4, 0, 1, 3)
            .reshape(batch_sz, hist, dim))


# speedup vs baseline: 1.8007x; 1.6682x over previous
import functools

import jax
import jax.numpy as jnp
from jax import lax
from jax.experimental import pallas as pl
from jax.experimental.pallas import tpu as pltpu
from jax.experimental.pallas import tpu_sc as plsc

NC = 2
NS = 16
NW = NC * NS
NB = 5
LANE = 128
SUB = 8


@functools.cache
def _build(batch_sz: int, hist: int, vocab: int, dim: int):
    assert batch_sz % (NW * LANE) == 0 and batch_sz // NW == LANE
    assert dim % SUB == 0
    bw = batch_sz // NW
    dt_n = dim // SUB
    assert hist % NB == 0 and hist >= 2 * NB
    n_grp = hist // NB

    def body(table_hbm, batch_hbm, out_hbm, idx_v, idx_t, *rest):
        bufs = rest[:NB]
        tbufs = rest[NB:2 * NB]
        gsems = rest[2 * NB:3 * NB]
        osems = rest[3 * NB:]
        K = NB - 1
        wid = lax.axis_index("s") * NC + lax.axis_index("c")

        pltpu.sync_copy(batch_hbm.at[pl.ds(wid * bw, bw)], idx_v)

        @pl.loop(0, hist)
        def _(h):
            cols = jnp.full((16,), 0, jnp.int32) + h
            for k in range(bw // 16):
                rows = lax.iota(jnp.int32, 16) + k * 16
                v = plsc.load_gather(idx_v, [rows, cols])
                idx_t[h, pl.ds(k * 16, 16)] = v

        def gather(h, slot):
            pltpu.async_copy(
                table_hbm.at[idx_t.at[h]], bufs[slot], gsems[slot])

        for j in range(K):
            gather(j, j)

        @pl.loop(0, n_grp)
        def _(g):
            for b in range(NB):
                j = g * NB + b
                pltpu.make_async_copy(
                    table_hbm.at[idx_t.at[b]], bufs[b], gsems[b]).wait()

                @pl.when(j + K < hist)
                def _():
                    gather(j + K, (b + K) % NB)

                @pl.when(g > 0)
                def _():
                    for dt in range(dt_n):
                        pltpu.make_async_copy(
                            tbufs[b].at[dt, :, :bw], out_hbm.at[0, dt, wid],
                            osems[b]).wait()

                @plsc.parallel_loop(0, bw)
                def _(r):
                    colv = jnp.full((16,), 0, jnp.int32) + r
                    for k in range(dim // 16):
                        dv = lax.iota(jnp.int32, 16) + k * 16
                        v = bufs[b][r, pl.ds(k * 16, 16)]
                        plsc.store_scatter(
                            tbufs[b], [dv // SUB, dv % SUB, colv], v)

                for dt in range(dt_n):
                    pltpu.async_copy(
                        tbufs[b].at[dt, :, :bw], out_hbm.at[j, dt, wid],
                        osems[b])

        for b in range(NB):
            for dt in range(dt_n):
                pltpu.make_async_copy(
                    tbufs[b].at[dt, :, :bw], out_hbm.at[0, dt, wid],
                    osems[b]).wait()

    return pl.kernel(
        body,
        out_type=jax.ShapeDtypeStruct(
            (hist, dt_n, NW, SUB, LANE), jnp.float32),
        mesh=plsc.VectorSubcoreMesh(core_axis_name="c", subcore_axis_name="s"),
        scratch_types=[
            pltpu.VMEM((bw, hist), jnp.int32),
            pltpu.VMEM((hist, bw), jnp.int32),
            *[pltpu.VMEM((bw, dim), jnp.float32) for _ in range(NB)],
            *[pltpu.VMEM((dt_n, SUB, bw + 1), jnp.float32) for _ in range(NB)],
            *[pltpu.SemaphoreType.DMA for _ in range(2 * NB)],
        ],
        compiler_params=pltpu.CompilerParams(
            use_tc_tiling_on_sc=False, needs_layout_passes=False),
    )


def kernel(batch, weight):
    batch_sz, hist = batch.shape
    vocab, dim = weight.shape
    t5 = _build(batch_sz, hist, vocab, dim)(weight, batch)
    return (t5.transpose(2, 4, 0, 1, 3)
            .reshape(batch_sz, hist, dim))


# speedup vs baseline: 1.8131x; 1.0069x over previous
import functools

import jax
import jax.numpy as jnp
from jax import lax
from jax.experimental import pallas as pl
from jax.experimental.pallas import tpu as pltpu
from jax.experimental.pallas import tpu_sc as plsc

NC = 2
NS = 16
NW = NC * NS
NB = 5
LANE = 128
SUB = 8


@functools.cache
def _build(batch_sz: int, hist: int, vocab: int, dim: int):
    assert batch_sz % (NW * LANE) == 0 and batch_sz // NW == LANE
    assert dim % SUB == 0
    bw = batch_sz // NW
    dt_n = dim // SUB
    assert hist % NB == 0 and hist >= 2 * NB
    n_grp = hist // NB

    def body(table_hbm, batch_hbm, out_hbm, idx_v, idx_t, *rest):
        bufs = rest[:NB]
        tbufs = rest[NB:2 * NB]
        gsems = rest[2 * NB:3 * NB]
        osems = rest[3 * NB:]
        K = NB - 1
        wid = lax.axis_index("s") * NC + lax.axis_index("c")

        pltpu.sync_copy(batch_hbm.at[pl.ds(wid * bw, bw)], idx_v)

        @pl.loop(0, hist)
        def _(h):
            cols = jnp.full((16,), 0, jnp.int32) + h
            for k in range(bw // 16):
                rows = lax.iota(jnp.int32, 16) + k * 16
                v = plsc.load_gather(idx_v, [rows, cols])
                idx_t[h, pl.ds(k * 16, 16)] = v

        def gather(h, slot):
            pltpu.async_copy(
                table_hbm.at[idx_t.at[h]], bufs[slot], gsems[slot])

        for j in range(K):
            gather(j, j)

        @pl.loop(0, n_grp)
        def _(g):
            for b in range(NB):
                j = g * NB + b
                pltpu.make_async_copy(
                    table_hbm.at[idx_t.at[b]], bufs[b], gsems[b]).wait()

                @pl.when(j + K < hist)
                def _():
                    gather(j + K, (b + K) % NB)

                @pl.when(g > 0)
                def _():
                    pltpu.make_async_copy(
                        tbufs[b].at[:, :, :bw], out_hbm.at[0, :, wid],
                        osems[b]).wait()

                @plsc.parallel_loop(0, bw, step=2)
                def _(r2):
                    for rr in range(2):
                        r = r2 + rr
                        colv = jnp.full((16,), 0, jnp.int32) + r
                        for k in range(dim // 16):
                            dv = lax.iota(jnp.int32, 16) + k * 16
                            v = bufs[b][r, pl.ds(k * 16, 16)]
                            plsc.store_scatter(
                                tbufs[b], [dv // SUB, dv % SUB, colv], v)

                pltpu.async_copy(
                    tbufs[b].at[:, :, :bw], out_hbm.at[j, :, wid], osems[b])

        for b in range(NB):
            pltpu.make_async_copy(
                tbufs[b].at[:, :, :bw], out_hbm.at[0, :, wid],
                osems[b]).wait()

    return pl.kernel(
        body,
        out_type=jax.ShapeDtypeStruct(
            (hist, dt_n, NW, SUB, LANE), jnp.float32),
        mesh=plsc.VectorSubcoreMesh(core_axis_name="c", subcore_axis_name="s"),
        scratch_types=[
            pltpu.VMEM((bw, hist), jnp.int32),
            pltpu.VMEM((hist, bw), jnp.int32),
            *[pltpu.VMEM((bw, dim), jnp.float32) for _ in range(NB)],
            *[pltpu.VMEM((dt_n, SUB, bw + 1), jnp.float32) for _ in range(NB)],
            *[pltpu.SemaphoreType.DMA for _ in range(2 * NB)],
        ],
        compiler_params=pltpu.CompilerParams(
            use_tc_tiling_on_sc=False, needs_layout_passes=False),
    )


def kernel(batch, weight):
    batch_sz, hist = batch.shape
    vocab, dim = weight.shape
    t5 = _build(batch_sz, hist, vocab, dim)(weight, batch)
    return (t5.transpose(2, 4, 0, 1, 3)
            .reshape(batch_sz, hist, dim))


# speedup vs baseline: 1.9412x; 1.0707x over previous
import functools

import jax
import jax.numpy as jnp
from jax import lax
from jax.experimental import pallas as pl
from jax.experimental.pallas import tpu as pltpu
from jax.experimental.pallas import tpu_sc as plsc

NC = 2
NS = 16
NW = NC * NS
NB = 5
LANE = 128
SUB = 8


@functools.cache
def _build(batch_sz: int, hist: int, vocab: int, dim: int):
    assert batch_sz % (NW * LANE) == 0 and batch_sz // NW == LANE
    assert dim % SUB == 0
    bw = batch_sz // NW
    dt_n = dim // SUB
    assert hist % NB == 0 and hist >= 2 * NB
    n_grp = hist // NB

    def body(table_hbm, batch_hbm, out_hbm, idx_v, idx_t, *rest):
        bufs = rest[:NB]
        tbufs = rest[NB:2 * NB]
        gsems = rest[2 * NB:3 * NB]
        osems = rest[3 * NB:]
        K = NB - 1
        wid = lax.axis_index("s") * NC + lax.axis_index("c")

        pltpu.sync_copy(batch_hbm.at[pl.ds(wid * bw, bw)], idx_v)

        @pl.loop(0, hist)
        def _(h):
            cols = jnp.full((16,), 0, jnp.int32) + h
            for k in range(bw // 16):
                rows = lax.iota(jnp.int32, 16) + k * 16
                v = plsc.load_gather(idx_v, [rows, cols])
                idx_t[h, pl.ds(k * 16, 16)] = v + v

        def gather(h, slot):
            pltpu.async_copy(
                table_hbm.at[idx_t.at[h]], bufs[slot], gsems[slot])

        for j in range(K):
            gather(j, j)

        @pl.loop(0, n_grp)
        def _(g):
            for b in range(NB):
                j = g * NB + b
                pltpu.make_async_copy(
                    table_hbm.at[idx_t.at[b]], bufs[b], gsems[b]).wait()

                @pl.when(j + K < hist)
                def _():
                    gather(j + K, (b + K) % NB)

                @pl.when(g > 0)
                def _():
                    pltpu.make_async_copy(
                        tbufs[b].at[:, :, :bw], out_hbm.at[0, :, wid],
                        osems[b]).wait()

                @plsc.parallel_loop(0, bw, step=2)
                def _(r2):
                    for rr in range(2):
                        r = r2 + rr
                        colv = jnp.full((16,), 0, jnp.int32) + r
                        for k in range(dim // 16):
                            dv = lax.iota(jnp.int32, 16) + k * 16
                            v = bufs[b][r, pl.ds(k * 16, 16)]
                            plsc.store_scatter(
                                tbufs[b], [dv // SUB, dv % SUB, colv], v)

                pltpu.async_copy(
                    tbufs[b].at[:, :, :bw], out_hbm.at[j, :, wid], osems[b])

        for b in range(NB):
            pltpu.make_async_copy(
                tbufs[b].at[:, :, :bw], out_hbm.at[0, :, wid],
                osems[b]).wait()

    return pl.kernel(
        body,
        name="emb_gather",
        out_type=jax.ShapeDtypeStruct(
            (hist, dt_n, NW, SUB, LANE), jnp.float32),
        mesh=plsc.VectorSubcoreMesh(core_axis_name="c", subcore_axis_name="s"),
        scratch_types=[
            pltpu.VMEM((bw, hist), jnp.int32),
            pltpu.VMEM((hist, bw), jnp.int32),
            *[pltpu.VMEM((bw, dim), jnp.float32) for _ in range(NB)],
            *[pltpu.VMEM((dt_n, SUB, bw + 1), jnp.float32) for _ in range(NB)],
            *[pltpu.SemaphoreType.DMA for _ in range(2 * NB)],
        ],
        compiler_params=pltpu.CompilerParams(
            use_tc_tiling_on_sc=False, needs_layout_passes=False),
    )


def kernel(batch, weight):
    batch_sz, hist = batch.shape
    vocab, dim = weight.shape
    table = jnp.pad(weight, ((0, 0), (0, LANE - dim))).reshape(2 * vocab, dim)
    t5 = _build(batch_sz, hist, vocab, dim)(table, batch)
    return (t5.transpose(2, 4, 0, 1, 3)
            .reshape(batch_sz, hist, dim))
